# speedup vs baseline: 1.5703x; 1.5703x over previous
import functools

import jax
import jax.numpy as jnp
from jax import lax
from jax.experimental import pallas as pl
from jax.experimental.pallas import tpu as pltpu
from jax.experimental.pallas import tpu_sc as plsc

_CHUNK = 128


@functools.cache
def _build(B, V, D):
    info = plsc.get_sparse_core_info()
    NC, NS = info.num_cores, info.num_subcores
    NW = NC * NS
    n_chunks = B // _CHUNK
    c_per_w = n_chunks // NW

    mesh = plsc.VectorSubcoreMesh(core_axis_name="c", subcore_axis_name="s")

    @functools.partial(
        pl.kernel,
        mesh=mesh,
        out_type=jax.ShapeDtypeStruct((n_chunks, _CHUNK, D), jnp.float32),
        scratch_types=[
            pltpu.VMEM((c_per_w, _CHUNK), jnp.int32),
            pltpu.VMEM((c_per_w, _CHUNK, D), jnp.float32),
            pltpu.SemaphoreType.DMA,
        ],
    )
    def k(idx_hbm, table_hbm, out_hbm, idx_v, rows_v, sem):
        wid = lax.axis_index("s") * NC + lax.axis_index("c")
        base = wid * c_per_w
        pltpu.sync_copy(idx_hbm.at[pl.ds(base, c_per_w)], idx_v)
        copies = [
            pltpu.async_copy(table_hbm.at[idx_v.at[j]], rows_v.at[j], sem)
            for j in range(c_per_w)
        ]
        for c in copies:
            c.wait()
        pltpu.sync_copy(rows_v, out_hbm.at[pl.ds(base, c_per_w)])

    return k


def kernel(x, table):
    B = x.shape[0]
    V, D = table.shape
    k = _build(B, V, D)
    idx2d = x.astype(jnp.int32).reshape(B // _CHUNK, _CHUNK)
    out = k(idx2d, table)
    return out.reshape(B, D)
